# baseline (speedup 1.0000x reference)
import jax
import jax.numpy as jnp
import numpy as np
from jax import lax
from jax.experimental import pallas as pl
from jax.experimental.pallas import tpu as pltpu
from jax.experimental.pallas import tpu_sc as plsc

_A = 32768
_NHEADS = 4
_ROWS = 128
_BLK = 2048

_CB = 10240
_NA = _A - _CB
_NB1 = _NA // _BLK
_NB2 = _CB // _BLK

_TINY = np.float32(np.finfo(np.float32).tiny)
_BIG_I32 = np.int32(2**31 - 1)


def _threefry_bits(k1, k2, x1):
  ks2 = k1 ^ k2 ^ np.uint32(0x1BD11BDA)
  ks = (k1, k2, ks2)
  rotations = ((13, 15, 26, 6), (17, 29, 16, 24))

  def rotl(v, d):
    return (v << np.uint32(d)) | (v >> np.uint32(32 - d))

  x0 = jnp.zeros_like(x1) + k1
  for i in range(5):
    for r in rotations[i % 2]:
      x0 = x0 + x1
      x1 = rotl(x1, r)
      x1 = x1 ^ x0
    x0 = x0 + ks[(i + 1) % 3]
    x1 = x1 + ks[(i + 2) % 3] + np.uint32(i + 1)
  return x0 ^ x1


def _gumbel_from_bits(bits):
  float_bits = (bits >> np.uint32(9)) | np.uint32(0x3F800000)
  f = pltpu.bitcast(float_bits, jnp.float32) - np.float32(1.0)
  u = jnp.maximum(f, _TINY)
  return jnp.log(-jnp.log(u))


def _tc1_kernel(keys_ref, x_ref, val_out, idx_out, best_val, best_idx, p_base):
  h = pl.program_id(0)
  cb = pl.program_id(1)

  k1 = keys_ref[h, 0]
  k2 = keys_ref[h, 1]

  @pl.when((h == 0) & (cb == 0))
  def _():
    row = lax.broadcasted_iota(jnp.uint32, (_ROWS, _BLK), 0)
    col = lax.broadcasted_iota(jnp.uint32, (_ROWS, _BLK), 1)
    p_base[...] = row * np.uint32(_A) + col

  x1 = p_base[...] + (k2 + (cb * _BLK).astype(jnp.uint32))
  bits = _threefry_bits(k1, k2, x1)
  v = x_ref[...] - _gumbel_from_bits(bits)

  m = jnp.max(v, axis=1, keepdims=True)
  colg = lax.broadcasted_iota(jnp.int32, (_ROWS, _BLK), 1) + cb * _BLK
  cand = jnp.where(v == m, colg, _BIG_I32)
  idx = jnp.min(cand, axis=1, keepdims=True)

  @pl.when(cb == 0)
  def _():
    best_val[...] = m
    best_idx[...] = idx

  @pl.when(cb != 0)
  def _():
    better = m > best_val[...]
    best_val[...] = jnp.where(better, m, best_val[...])
    best_idx[...] = jnp.where(better, idx, best_idx[...])

  @pl.when(cb == _NB1 - 1)
  def _():
    lane = lax.broadcasted_iota(jnp.int32, (_ROWS, _NHEADS), 1)
    val_out[...] = jnp.where(lane == h, best_val[...], val_out[...])
    idx_out[...] = jnp.where(lane == h, best_idx[...], idx_out[...])


def _sc_bits_kernel(keys_hbm, bits_hbm, kv, buf):
  wid = lax.axis_index("s") * 2 + lax.axis_index("c")
  pltpu.sync_copy(keys_hbm, kv)

  lane = lax.iota(jnp.int32, 16).astype(jnp.uint32)

  def stripe(t, carry):
    hr = wid * 16 + t
    h = hr // _ROWS
    r = hr - h * _ROWS
    k1v = kv[h, 0, :]
    k2v = kv[h, 1, :]
    base = (r * _A + _NA).astype(jnp.uint32)

    def vec(j, c):
      p = lane + (base + (j * 16).astype(jnp.uint32))
      bits = _threefry_bits(k1v, k2v, p + k2v)
      buf[pl.ds(pl.multiple_of(j * 16, 16), 16)] = bits
      return c

    lax.fori_loop(0, _CB // 16, vec, 0, unroll=8)
    pltpu.sync_copy(buf, bits_hbm.at[hr, :])
    return carry

  lax.fori_loop(0, 16, stripe, 0)


def _tc2_kernel(pval_ref, pidx_ref, bits_ref, x_ref, out_ref, s_val, s_idx):
  h = pl.program_id(0)
  cb = pl.program_id(1)

  @pl.when((h == 0) & (cb == 0))
  def _():
    s_val[...] = pval_ref[...]
    s_idx[...] = pidx_ref[...]

  v = x_ref[...] - _gumbel_from_bits(bits_ref[...])

  m = jnp.max(v, axis=1, keepdims=True)
  colg = (lax.broadcasted_iota(jnp.int32, (_ROWS, _BLK), 1)
          + (_NA + cb * _BLK))
  cand = jnp.where(v == m, colg, _BIG_I32)
  idx = jnp.min(cand, axis=1, keepdims=True)

  lane = lax.broadcasted_iota(jnp.int32, (_ROWS, _NHEADS), 1)
  better = (lane == h) & (m > s_val[...])
  s_val[...] = jnp.where(better, jnp.broadcast_to(m, (_ROWS, _NHEADS)),
                         s_val[...])
  s_idx[...] = jnp.where(better, jnp.broadcast_to(idx, (_ROWS, _NHEADS)),
                         s_idx[...])

  @pl.when((h == _NHEADS - 1) & (cb == _NB2 - 1))
  def _():
    out_ref[...] = s_idx[...]


@jax.jit
def kernel(x):
  base = jax.random.key(42)
  keys = jnp.stack(
      [jax.random.key_data(jax.random.fold_in(base, i)) for i in range(_NHEADS)]
  ).astype(jnp.uint32)
  keys_exp = jnp.broadcast_to(keys[:, :, None], (_NHEADS, 2, 16))

  sc_bits = pl.kernel(
      _sc_bits_kernel,
      out_type=jax.ShapeDtypeStruct((_NHEADS * _ROWS, _CB), jnp.uint32),
      mesh=plsc.VectorSubcoreMesh(core_axis_name="c", subcore_axis_name="s"),
      scratch_types=[
          pltpu.VMEM((_NHEADS, 2, 16), jnp.uint32),
          pltpu.VMEM((_CB,), jnp.uint32),
      ],
  )(keys_exp)

  pval, pidx = pl.pallas_call(
      _tc1_kernel,
      grid=(_NHEADS, _NB1),
      in_specs=[
          pl.BlockSpec(memory_space=pltpu.SMEM),
          pl.BlockSpec((_ROWS, _BLK), lambda h, cb: (0, h * (_A // _BLK) + cb)),
      ],
      out_specs=[
          pl.BlockSpec((_ROWS, _NHEADS), lambda h, cb: (0, 0)),
          pl.BlockSpec((_ROWS, _NHEADS), lambda h, cb: (0, 0)),
      ],
      out_shape=[
          jax.ShapeDtypeStruct((_ROWS, _NHEADS), jnp.float32),
          jax.ShapeDtypeStruct((_ROWS, _NHEADS), jnp.int32),
      ],
      scratch_shapes=[
          pltpu.VMEM((_ROWS, 1), jnp.float32),
          pltpu.VMEM((_ROWS, 1), jnp.int32),
          pltpu.VMEM((_ROWS, _BLK), jnp.uint32),
      ],
      compiler_params=pltpu.CompilerParams(
          dimension_semantics=("arbitrary", "arbitrary"),
      ),
  )(keys, x)

  out = pl.pallas_call(
      _tc2_kernel,
      grid=(_NHEADS, _NB2),
      in_specs=[
          pl.BlockSpec((_ROWS, _NHEADS), lambda h, cb: (0, 0)),
          pl.BlockSpec((_ROWS, _NHEADS), lambda h, cb: (0, 0)),
          pl.BlockSpec((_ROWS, _BLK), lambda h, cb: (h, cb)),
          pl.BlockSpec(
              (_ROWS, _BLK),
              lambda h, cb: (0, h * (_A // _BLK) + (_NA // _BLK) + cb)),
      ],
      out_specs=pl.BlockSpec((_ROWS, _NHEADS), lambda h, cb: (0, 0)),
      out_shape=jax.ShapeDtypeStruct((_ROWS, _NHEADS), jnp.int32),
      scratch_shapes=[
          pltpu.VMEM((_ROWS, _NHEADS), jnp.float32),
          pltpu.VMEM((_ROWS, _NHEADS), jnp.int32),
      ],
      compiler_params=pltpu.CompilerParams(
          dimension_semantics=("arbitrary", "arbitrary"),
      ),
  )(pval, pidx, sc_bits, x)

  return out.T.reshape(-1)


# speedup vs baseline: 1.0477x; 1.0477x over previous
import jax
import jax.numpy as jnp
import numpy as np
from jax import lax
from jax.experimental import pallas as pl
from jax.experimental.pallas import tpu as pltpu
from jax.experimental.pallas import tpu_sc as plsc

_A = 32768
_NHEADS = 4
_ROWS = 128
_BLK = 2048

_CB = 8192
_NA = _A - _CB
_NB1 = _NA // _BLK
_BLK2 = 8192
_NB2 = _CB // _BLK2

_TINY = np.float32(np.finfo(np.float32).tiny)
_BIG_I32 = np.int32(2**31 - 1)


def _threefry_bits(k1, k2, x1):
  ks2 = k1 ^ k2 ^ np.uint32(0x1BD11BDA)
  ks = (k1, k2, ks2)
  rotations = ((13, 15, 26, 6), (17, 29, 16, 24))

  def rotl(v, d):
    return (v << np.uint32(d)) | (v >> np.uint32(32 - d))

  x0 = jnp.zeros_like(x1) + k1
  for i in range(5):
    for r in rotations[i % 2]:
      x0 = x0 + x1
      x1 = rotl(x1, r)
      x1 = x1 ^ x0
    x0 = x0 + ks[(i + 1) % 3]
    x1 = x1 + ks[(i + 2) % 3] + np.uint32(i + 1)
  return x0 ^ x1


def _gumbel_from_bits(bits):
  float_bits = (bits >> np.uint32(9)) | np.uint32(0x3F800000)
  f = pltpu.bitcast(float_bits, jnp.float32) - np.float32(1.0)
  u = jnp.maximum(f, _TINY)
  return jnp.log(-jnp.log(u))


def _tc1_kernel(keys_ref, x_ref, val_out, idx_out, best_val, best_idx, p_base):
  h = pl.program_id(0)
  cb = pl.program_id(1)

  k1 = keys_ref[h, 0]
  k2 = keys_ref[h, 1]

  @pl.when((h == 0) & (cb == 0))
  def _():
    row = lax.broadcasted_iota(jnp.uint32, (_ROWS, _BLK), 0)
    col = lax.broadcasted_iota(jnp.uint32, (_ROWS, _BLK), 1)
    p_base[...] = row * np.uint32(_A) + col

  x1 = p_base[...] + (k2 + (cb * _BLK).astype(jnp.uint32))
  bits = _threefry_bits(k1, k2, x1)
  v = x_ref[...] - _gumbel_from_bits(bits)

  m = jnp.max(v, axis=1, keepdims=True)
  colg = lax.broadcasted_iota(jnp.int32, (_ROWS, _BLK), 1) + cb * _BLK
  cand = jnp.where(v == m, colg, _BIG_I32)
  idx = jnp.min(cand, axis=1, keepdims=True)

  @pl.when(cb == 0)
  def _():
    best_val[...] = m
    best_idx[...] = idx

  @pl.when(cb != 0)
  def _():
    better = m > best_val[...]
    best_val[...] = jnp.where(better, m, best_val[...])
    best_idx[...] = jnp.where(better, idx, best_idx[...])

  @pl.when(cb == _NB1 - 1)
  def _():
    lane = lax.broadcasted_iota(jnp.int32, (_ROWS, _NHEADS), 1)
    val_out[...] = jnp.where(lane == h, best_val[...], val_out[...])
    idx_out[...] = jnp.where(lane == h, best_idx[...], idx_out[...])


def _sc_bits_kernel(keys_hbm, bits_hbm, kv, buf):
  wid = lax.axis_index("s") * 2 + lax.axis_index("c")
  pltpu.sync_copy(keys_hbm, kv)

  lane = lax.iota(jnp.int32, 16).astype(jnp.uint32)

  def stripe(t, carry):
    hr = wid * 16 + t
    h = hr // _ROWS
    r = hr - h * _ROWS
    k1v = kv[h, 0, :]
    k2v = kv[h, 1, :]
    base = (r * _A + _NA).astype(jnp.uint32)

    def vec(j, c):
      p = lane + (base + (j * 16).astype(jnp.uint32))
      bits = _threefry_bits(k1v, k2v, p + k2v)
      buf[pl.ds(pl.multiple_of(j * 16, 16), 16)] = bits
      return c

    lax.fori_loop(0, _CB // 16, vec, 0, unroll=8)
    pltpu.sync_copy(buf, bits_hbm.at[hr, :])
    return carry

  lax.fori_loop(0, 16, stripe, 0)


def _tc2_kernel(pval_ref, pidx_ref, bits_ref, x_ref, out_ref, s_val, s_idx):
  h = pl.program_id(0)
  cb = pl.program_id(1)

  @pl.when((h == 0) & (cb == 0))
  def _():
    s_val[...] = pval_ref[...]
    s_idx[...] = pidx_ref[...]

  v = x_ref[...] - _gumbel_from_bits(bits_ref[...])

  m = jnp.max(v, axis=1, keepdims=True)
  colg = (lax.broadcasted_iota(jnp.int32, (_ROWS, _BLK2), 1)
          + (_NA + cb * _BLK2))
  cand = jnp.where(v == m, colg, _BIG_I32)
  idx = jnp.min(cand, axis=1, keepdims=True)

  lane = lax.broadcasted_iota(jnp.int32, (_ROWS, _NHEADS), 1)
  better = (lane == h) & (m > s_val[...])
  s_val[...] = jnp.where(better, jnp.broadcast_to(m, (_ROWS, _NHEADS)),
                         s_val[...])
  s_idx[...] = jnp.where(better, jnp.broadcast_to(idx, (_ROWS, _NHEADS)),
                         s_idx[...])

  @pl.when((h == _NHEADS - 1) & (cb == _NB2 - 1))
  def _():
    out_ref[...] = s_idx[...].T


@jax.jit
def kernel(x):
  base = jax.random.key(42)
  keys = jnp.stack(
      [jax.random.key_data(jax.random.fold_in(base, i)) for i in range(_NHEADS)]
  ).astype(jnp.uint32)
  keys_exp = jnp.broadcast_to(keys[:, :, None], (_NHEADS, 2, 16))

  sc_bits = pl.kernel(
      _sc_bits_kernel,
      out_type=jax.ShapeDtypeStruct((_NHEADS * _ROWS, _CB), jnp.uint32),
      mesh=plsc.VectorSubcoreMesh(core_axis_name="c", subcore_axis_name="s"),
      scratch_types=[
          pltpu.VMEM((_NHEADS, 2, 16), jnp.uint32),
          pltpu.VMEM((_CB,), jnp.uint32),
      ],
  )(keys_exp)

  pval, pidx = pl.pallas_call(
      _tc1_kernel,
      grid=(_NHEADS, _NB1),
      in_specs=[
          pl.BlockSpec(memory_space=pltpu.SMEM),
          pl.BlockSpec((_ROWS, _BLK), lambda h, cb: (0, h * (_A // _BLK) + cb)),
      ],
      out_specs=[
          pl.BlockSpec((_ROWS, _NHEADS), lambda h, cb: (0, 0)),
          pl.BlockSpec((_ROWS, _NHEADS), lambda h, cb: (0, 0)),
      ],
      out_shape=[
          jax.ShapeDtypeStruct((_ROWS, _NHEADS), jnp.float32),
          jax.ShapeDtypeStruct((_ROWS, _NHEADS), jnp.int32),
      ],
      scratch_shapes=[
          pltpu.VMEM((_ROWS, 1), jnp.float32),
          pltpu.VMEM((_ROWS, 1), jnp.int32),
          pltpu.VMEM((_ROWS, _BLK), jnp.uint32),
      ],
      compiler_params=pltpu.CompilerParams(
          dimension_semantics=("arbitrary", "arbitrary"),
      ),
  )(keys, x)

  out = pl.pallas_call(
      _tc2_kernel,
      grid=(_NHEADS, _NB2),
      in_specs=[
          pl.BlockSpec((_ROWS, _NHEADS), lambda h, cb: (0, 0)),
          pl.BlockSpec((_ROWS, _NHEADS), lambda h, cb: (0, 0)),
          pl.BlockSpec((_ROWS, _BLK2), lambda h, cb: (h, cb)),
          pl.BlockSpec(
              (_ROWS, _BLK2),
              lambda h, cb: (0, h * (_A // _BLK2) + (_NA // _BLK2) + cb)),
      ],
      out_specs=pl.BlockSpec((_NHEADS, _ROWS), lambda h, cb: (0, 0)),
      out_shape=jax.ShapeDtypeStruct((_NHEADS, _ROWS), jnp.int32),
      scratch_shapes=[
          pltpu.VMEM((_ROWS, _NHEADS), jnp.float32),
          pltpu.VMEM((_ROWS, _NHEADS), jnp.int32),
      ],
      compiler_params=pltpu.CompilerParams(
          dimension_semantics=("arbitrary", "arbitrary"),
      ),
  )(pval, pidx, sc_bits, x)

  return out.reshape(-1)
